# Initial kernel scaffold; baseline (speedup 1.0000x reference)
import jax
import jax.numpy as jnp
from jax.experimental import pallas as pl


def kernel(x, W1, b1, W2, b2, Wf1, bf1, Wf2, bf2):
    raise NotImplementedError("write your pallas kernel here")


# speedup vs baseline: 1.5601x; 1.5601x over previous
import math

import jax
import jax.numpy as jnp
from jax.experimental import pallas as pl
from jax.experimental.pallas import tpu as pltpu

_B = 4
_D = 3
_DP = 8
_V = 2048
_KNN = 20
_K1 = 5
_F1 = 128
_K2 = 5
_F2 = 256
_FEAT1 = 512
_FEAT2 = 128

_RB = 256
_NRB = _V // _RB

_HI = jax.lax.Precision.HIGHEST


def _dot(a, b):
    return jax.lax.dot_general(a, b, (((1,), (0,)), ((), ())),
                               preferred_element_type=jnp.float32,
                               precision=_HI)


def _gcn_kernel(x_ref, w1_ref, b1_ref, w2_ref, b2_ref, wf1_ref, bf1_ref,
                wf2_ref, bf2_ref, o_ref, s_ref, a_ref, rs_ref, t_ref):
    xb = x_ref[0]
    sq = jnp.sum(xb * xb, axis=0, keepdims=True)

    def pass1(i, carry):
        xblk = x_ref[0, :, pl.ds(i * _RB, _RB)]
        sqb = jnp.sum(xblk * xblk, axis=0, keepdims=True)
        gram = jax.lax.dot_general(xblk, xb, (((0,), (0,)), ((), ())),
                                   preferred_element_type=jnp.float32,
                                   precision=_HI)
        d2 = jnp.maximum(sqb.T + sq - 2.0 * gram, 0.0)
        distb = jnp.sqrt(d2)
        row_i = jax.lax.broadcasted_iota(jnp.int32, (_RB, _V), 0) + i * _RB
        col_i = jax.lax.broadcasted_iota(jnp.int32, (_RB, _V), 1)
        blk = jnp.where(row_i == col_i, jnp.inf, distb)
        work = blk
        m = None
        for _ in range(_KNN):
            m = jnp.min(work, axis=1, keepdims=True)
            work = jnp.where(work == m, jnp.inf, work)
        thr = m
        mask = (blk <= thr) & (blk > 0.0)
        maskf = mask.astype(jnp.float32)
        cnt = jnp.sum(maskf, axis=1, keepdims=True)
        sigma = jnp.sum(blk * maskf, axis=1, keepdims=True) / cnt
        wgt = jnp.exp(-(blk * blk) / (sigma * sigma))
        wgt = jnp.where(mask, wgt, 0.0)
        s_ref[pl.ds(i * _RB, _RB), :] = wgt
        rs_ref[pl.ds(i * _RB, _RB), :] = jnp.sum(wgt, axis=1, keepdims=True)
        return carry

    jax.lax.fori_loop(0, _NRB, pass1, 0, unroll=False)

    rs = rs_ref[...]
    dis0 = rs ** -0.5
    dis = jnp.where(jnp.isinf(dis0), 0.0, dis0)

    def pass2(i, carry):
        wgtb = s_ref[pl.ds(i * _RB, _RB), :]
        rsb = rs_ref[pl.ds(i * _RB, _RB), :]
        disb0 = rsb ** -0.5
        disb = jnp.where(jnp.isinf(disb0), 0.0, disb0)
        blk_t = jnp.transpose(wgtb)
        a_ref[:, pl.ds(i * _RB, _RB)] = dis * blk_t * jnp.transpose(disb)
        return carry

    jax.lax.fori_loop(0, _NRB, pass2, 0, unroll=False)

    def amul(x_val, w):
        def body(i, carry):
            ab = a_ref[pl.ds(i * _RB, _RB), :]
            t_ref[pl.ds(i * _RB, _RB), pl.ds(0, w)] = _dot(ab, x_val)
            return carry
        jax.lax.fori_loop(0, _NRB, body, 0, unroll=False)
        return t_ref[:, pl.ds(0, w)]

    def lmul(x_val, w):
        return x_val - amul(x_val, w)

    v0 = jnp.full((_V, 1), 1.0 / math.sqrt(float(_V)), dtype=jnp.float32)

    def piter(t, vv):
        w = lmul(vv, 1)
        return w / jnp.sqrt(jnp.sum(w * w))

    v = jax.lax.fori_loop(0, 100, piter, v0, unroll=False)
    lmax = jnp.sum(v * lmul(v, 1))
    alpha = 2.0 / lmax

    def lr_mul(x_val, w):
        return alpha * lmul(x_val, w) - x_val

    def cheby(x0, ws_ref, bias, k_order, w):
        y = bias + _dot(x0, ws_ref[0])
        xkm2, xkm1 = x0, None
        for k in range(1, k_order):
            if k == 1:
                xk = lr_mul(x0, w)
            else:
                xk = 2.0 * lr_mul(xkm1, w) - xkm2
                xkm2 = xkm1
            xkm1 = xk
            y = y + _dot(xk, ws_ref[k])
        return y

    x0 = jnp.transpose(xb)
    y1 = jax.nn.relu(cheby(x0, w1_ref, b1_ref[...], _K1, _DP))
    y2 = jax.nn.relu(cheby(y1, w2_ref, b2_ref[...], _K2, _F1))
    e1 = jax.nn.relu(_dot(y2, wf1_ref[...]) + bf1_ref[...])
    e2 = jax.nn.relu(_dot(e1, wf2_ref[...]) + bf2_ref[...])
    o_ref[0] = e2


@jax.jit
def kernel(x, W1, b1, W2, b2, Wf1, bf1, Wf2, bf2):
    xp = jnp.concatenate(
        [x, jnp.zeros((_B, _DP - _D, _V), jnp.float32)], axis=1)
    w1s = jnp.transpose(W1.reshape(_F1, _D, _K1), (2, 1, 0))
    w1s = jnp.concatenate(
        [w1s, jnp.zeros((_K1, _DP - _D, _F1), jnp.float32)], axis=1)
    w2s = jnp.transpose(W2.reshape(_F2, _F1, _K2), (2, 1, 0))
    wf1t = jnp.transpose(Wf1)
    wf2t = jnp.transpose(Wf2)
    b1r = b1.reshape(1, _F1)
    b2r = b2.reshape(1, _F2)
    bf1r = bf1.reshape(1, _FEAT1)
    bf2r = bf2.reshape(1, _FEAT2)

    full = lambda shp: pl.BlockSpec(shp, lambda b: (0,) * len(shp))
    out = pl.pallas_call(
        _gcn_kernel,
        grid=(_B,),
        in_specs=[
            pl.BlockSpec((1, _DP, _V), lambda b: (b, 0, 0)),
            full(w1s.shape), full(b1r.shape),
            full(w2s.shape), full(b2r.shape),
            full(wf1t.shape), full(bf1r.shape),
            full(wf2t.shape), full(bf2r.shape),
        ],
        out_specs=pl.BlockSpec((1, _V, _FEAT2), lambda b: (b, 0, 0)),
        out_shape=jax.ShapeDtypeStruct((_B, _V, _FEAT2), jnp.float32),
        scratch_shapes=[
            pltpu.VMEM((_V, _V), jnp.float32),
            pltpu.VMEM((_V, _V), jnp.float32),
            pltpu.VMEM((_V, 1), jnp.float32),
            pltpu.VMEM((_V, _F1), jnp.float32),
        ],
    )(xp, w1s, b1r, w2s, b2r, wf1t, bf1r, wf2t, bf2r)
    return out


# speedup vs baseline: 6.5250x; 4.1824x over previous
import math

import jax
import jax.numpy as jnp
from jax.experimental import pallas as pl
from jax.experimental.pallas import tpu as pltpu

_B = 4
_D = 3
_DP = 8
_V = 2048
_KNN = 20
_K1 = 5
_F1 = 128
_K2 = 5
_F2 = 256
_FEAT1 = 512
_FEAT2 = 128

_RB = 128
_NRB = _V // _RB
_SB = 256
_NSB = _V // _SB

_HI = jax.lax.Precision.HIGHEST
_SQP = jax.lax.Precision.HIGHEST


def _dot(a, b):
    return jax.lax.dot_general(a, b, (((1,), (0,)), ((), ())),
                               preferred_element_type=jnp.float32,
                               precision=_HI)


def _gcn_kernel(x_ref, w1_ref, b1_ref, w2_ref, b2_ref, wf1_ref, bf1_ref,
                wf2_ref, bf2_ref, o_ref, s_ref, a_ref, m_ref, t_ref):
    xb = x_ref[0]
    sq = jnp.sum(xb * xb, axis=0, keepdims=True)

    def pass1(i, carry):
        xblk = x_ref[0, :, pl.ds(i * _RB, _RB)]
        sqb = jnp.sum(xblk * xblk, axis=0, keepdims=True)
        gram = jax.lax.dot_general(xblk, xb, (((0,), (0,)), ((), ())),
                                   preferred_element_type=jnp.float32,
                                   precision=_HI)
        d2 = jnp.maximum(sqb.T + sq - 2.0 * gram, 0.0)
        distb = jnp.sqrt(d2)
        row_i = jax.lax.broadcasted_iota(jnp.int32, (_RB, _V), 0) + i * _RB
        col_i = jax.lax.broadcasted_iota(jnp.int32, (_RB, _V), 1)
        blk = jnp.where(row_i == col_i, jnp.inf, distb)
        work = blk
        m = None
        for _ in range(_KNN):
            m = jnp.min(work, axis=1, keepdims=True)
            work = jnp.where(work == m, jnp.inf, work)
        thr = m
        mask = (blk <= thr) & (blk > 0.0)
        maskf = mask.astype(jnp.float32)
        cnt = jnp.sum(maskf, axis=1, keepdims=True)
        sigma = jnp.sum(blk * maskf, axis=1, keepdims=True) / cnt
        wgt = jnp.exp(-(blk * blk) / (sigma * sigma))
        wgt = jnp.where(mask, wgt, 0.0)
        s_ref[pl.ds(i * _RB, _RB), :] = wgt
        t_ref[pl.ds(i * _RB, _RB), pl.ds(0, 1)] = jnp.sum(
            wgt, axis=1, keepdims=True)
        return carry

    jax.lax.fori_loop(0, _NRB, pass1, 0, unroll=False)

    rs = t_ref[:, pl.ds(0, 1)]
    dis0 = rs ** -0.5
    dis = jnp.where(jnp.isinf(dis0), 0.0, dis0)

    def pass2(i, carry):
        wgtb = s_ref[pl.ds(i * _RB, _RB), :]
        rsb = t_ref[pl.ds(i * _RB, _RB), pl.ds(0, 1)]
        disb0 = rsb ** -0.5
        disb = jnp.where(jnp.isinf(disb0), 0.0, disb0)
        blk_t = jnp.transpose(wgtb)
        row_i = jax.lax.broadcasted_iota(jnp.int32, (_V, _RB), 0)
        col_i = jax.lax.broadcasted_iota(jnp.int32, (_V, _RB), 1) + i * _RB
        eye = jnp.where(row_i == col_i, 1.0, 0.0)
        a_ref[:, pl.ds(i * _RB, _RB)] = eye - dis * blk_t * jnp.transpose(disb)
        return carry

    jax.lax.fori_loop(0, _NRB, pass2, 0, unroll=False)

    def lmul(x_val, w):
        def body(i, carry):
            ab = a_ref[pl.ds(i * _RB, _RB), :]
            t_ref[pl.ds(i * _RB, _RB), pl.ds(0, w)] = _dot(ab, x_val)
            return carry
        jax.lax.fori_loop(0, _NRB, body, 0, unroll=False)
        return t_ref[:, pl.ds(0, w)]

    def sq_mm(src_ref, dst_ref, scale):
        def body_i(i, c0):
            def body_j(j, c1):
                acc = jnp.zeros((_SB, _SB), jnp.float32)
                for k in range(_NSB):
                    a = src_ref[pl.ds(i * _SB, _SB), pl.ds(k * _SB, _SB)]
                    b = src_ref[pl.ds(k * _SB, _SB), pl.ds(j * _SB, _SB)]
                    acc = acc + jax.lax.dot_general(
                        a.astype(jnp.bfloat16), b.astype(jnp.bfloat16),
                        (((1,), (0,)), ((), ())),
                        preferred_element_type=jnp.float32)
                dst_ref[pl.ds(i * _SB, _SB), pl.ds(j * _SB, _SB)] = (
                    scale * acc).astype(dst_ref.dtype)
                return c1
            jax.lax.fori_loop(0, _NSB, body_j, 0, unroll=False)
            return c0
        jax.lax.fori_loop(0, _NSB, body_i, 0, unroll=False)

    def mv(src_ref, x_val):
        xb16 = x_val.astype(jnp.bfloat16)
        def body(i, carry):
            ab = src_ref[pl.ds(i * _RB, _RB), :]
            t_ref[pl.ds(i * _RB, _RB), pl.ds(0, 1)] = jax.lax.dot_general(
                ab.astype(jnp.bfloat16), xb16, (((1,), (0,)), ((), ())),
                preferred_element_type=jnp.float32)
            return carry
        jax.lax.fori_loop(0, _NRB, body, 0, unroll=False)
        return t_ref[:, pl.ds(0, 1)]

    def normed(x_val):
        return x_val / jnp.sqrt(jnp.sum(x_val * x_val))

    u = jnp.full((_V, 1), 1.0 / math.sqrt(float(_V)), dtype=jnp.float32)
    sq_mm(a_ref, m_ref, 0.25)
    sq_mm(m_ref, s_ref, 1.0)
    u = normed(mv(s_ref, u))
    sq_mm(s_ref, m_ref, 1.0)
    sq_mm(m_ref, s_ref, 1.0)
    sq_mm(s_ref, m_ref, 1.0)
    u = normed(mv(m_ref, u))
    u = normed(mv(m_ref, u))
    u = normed(mv(m_ref, u))
    v = u
    lmax = jnp.sum(v * lmul(v, 1))
    alpha = 2.0 / lmax

    def lr_mul(x_val, w):
        return alpha * lmul(x_val, w) - x_val

    def cheby(x0, ws_ref, bias, k_order, w):
        y = bias + _dot(x0, ws_ref[0])
        xkm2, xkm1 = x0, None
        for k in range(1, k_order):
            if k == 1:
                xk = lr_mul(x0, w)
            else:
                xk = 2.0 * lr_mul(xkm1, w) - xkm2
                xkm2 = xkm1
            xkm1 = xk
            y = y + _dot(xk, ws_ref[k])
        return y

    x0 = jnp.transpose(xb)
    y1 = jax.nn.relu(cheby(x0, w1_ref, b1_ref[...], _K1, _DP))
    y2 = jax.nn.relu(cheby(y1, w2_ref, b2_ref[...], _K2, _F1))

    for i in range(_NRB):
        y2b = y2[i * _RB:(i + 1) * _RB, :]
        e1 = jax.nn.relu(_dot(y2b, wf1_ref[...]) + bf1_ref[...])
        e2 = jax.nn.relu(_dot(e1, wf2_ref[...]) + bf2_ref[...])
        o_ref[0, pl.ds(i * _RB, _RB), :] = e2


@jax.jit
def kernel(x, W1, b1, W2, b2, Wf1, bf1, Wf2, bf2):
    xp = jnp.concatenate(
        [x, jnp.zeros((_B, _DP - _D, _V), jnp.float32)], axis=1)
    w1s = jnp.transpose(W1.reshape(_F1, _D, _K1), (2, 1, 0))
    w1s = jnp.concatenate(
        [w1s, jnp.zeros((_K1, _DP - _D, _F1), jnp.float32)], axis=1)
    w2s = jnp.transpose(W2.reshape(_F2, _F1, _K2), (2, 1, 0))
    wf1t = jnp.transpose(Wf1)
    wf2t = jnp.transpose(Wf2)
    b1r = b1.reshape(1, _F1)
    b2r = b2.reshape(1, _F2)
    bf1r = bf1.reshape(1, _FEAT1)
    bf2r = bf2.reshape(1, _FEAT2)

    full = lambda shp: pl.BlockSpec(shp, lambda b: (0,) * len(shp))
    out = pl.pallas_call(
        _gcn_kernel,
        grid=(_B,),
        in_specs=[
            pl.BlockSpec((1, _DP, _V), lambda b: (b, 0, 0)),
            full(w1s.shape), full(b1r.shape),
            full(w2s.shape), full(b2r.shape),
            full(wf1t.shape), full(bf1r.shape),
            full(wf2t.shape), full(bf2r.shape),
        ],
        out_specs=pl.BlockSpec((1, _V, _FEAT2), lambda b: (b, 0, 0)),
        out_shape=jax.ShapeDtypeStruct((_B, _V, _FEAT2), jnp.float32),
        scratch_shapes=[
            pltpu.VMEM((_V, _V), jnp.float32),
            pltpu.VMEM((_V, _V), jnp.float32),
            pltpu.VMEM((_V, _V), jnp.bfloat16),
            pltpu.VMEM((_V, _F1), jnp.float32),
        ],
    )(xp, w1s, b1r, w2s, b2r, wf1t, bf1r, wf2t, bf2r)
    return out


# speedup vs baseline: 7.3665x; 1.1290x over previous
import math

import jax
import jax.numpy as jnp
from jax.experimental import pallas as pl
from jax.experimental.pallas import tpu as pltpu

_B = 4
_D = 3
_DP = 8
_V = 2048
_KNN = 20
_K1 = 5
_F1 = 128
_K2 = 5
_F2 = 256
_FEAT1 = 512
_FEAT2 = 128

_RB = 128
_NRB = _V // _RB
_SB = 256
_NSB = _V // _SB

_HI = jax.lax.Precision.HIGHEST


def _dot(a, b, prec=_HI):
    return jax.lax.dot_general(a, b, (((1,), (0,)), ((), ())),
                               preferred_element_type=jnp.float32,
                               precision=prec)


def _gcn_kernel(x_ref, w1_ref, b1_ref, w2_ref, b2_ref, wf1_ref, bf1_ref,
                wf2_ref, bf2_ref, o_ref, s_ref, ah_ref, al_ref, m_ref, t_ref):
    xb = x_ref[0]
    sq = jnp.sum(xb * xb, axis=0, keepdims=True)

    def pass1(i, carry):
        xblk = x_ref[0, :, pl.ds(i * _RB, _RB)]
        sqb = jnp.sum(xblk * xblk, axis=0, keepdims=True)
        gram = jax.lax.dot_general(xblk, xb, (((0,), (0,)), ((), ())),
                                   preferred_element_type=jnp.float32,
                                   precision=_HI)
        d2 = jnp.maximum(sqb.T + sq - 2.0 * gram, 0.0)
        distb = jnp.sqrt(d2)
        row_i = jax.lax.broadcasted_iota(jnp.int32, (_RB, _V), 0) + i * _RB
        col_i = jax.lax.broadcasted_iota(jnp.int32, (_RB, _V), 1)
        blk = jnp.where(row_i == col_i, jnp.inf, distb)
        work = blk
        m = None
        for _ in range(_KNN):
            m = jnp.min(work, axis=1, keepdims=True)
            work = jnp.where(work == m, jnp.inf, work)
        thr = m
        mask = (blk <= thr) & (blk > 0.0)
        maskf = mask.astype(jnp.float32)
        cnt = jnp.sum(maskf, axis=1, keepdims=True)
        sigma = jnp.sum(blk * maskf, axis=1, keepdims=True) / cnt
        wgt = jnp.exp(-(blk * blk) / (sigma * sigma))
        wgt = jnp.where(mask, wgt, 0.0)
        s_ref[pl.ds(i * _RB, _RB), :] = wgt
        t_ref[pl.ds(i * _RB, _RB), pl.ds(0, 1)] = jnp.sum(
            wgt, axis=1, keepdims=True)
        return carry

    jax.lax.fori_loop(0, _NRB, pass1, 0, unroll=False)

    rs = t_ref[:, pl.ds(0, 1)]
    dis0 = rs ** -0.5
    dis = jnp.where(jnp.isinf(dis0), 0.0, dis0)

    def pass2(i, carry):
        wgtb = s_ref[pl.ds(i * _RB, _RB), :]
        rsb = t_ref[pl.ds(i * _RB, _RB), pl.ds(0, 1)]
        disb0 = rsb ** -0.5
        disb = jnp.where(jnp.isinf(disb0), 0.0, disb0)
        blk_t = jnp.transpose(wgtb)
        row_i = jax.lax.broadcasted_iota(jnp.int32, (_V, _RB), 0)
        col_i = jax.lax.broadcasted_iota(jnp.int32, (_V, _RB), 1) + i * _RB
        eye = jnp.where(row_i == col_i, 1.0, 0.0)
        lb = eye - dis * blk_t * jnp.transpose(disb)
        hi = lb.astype(jnp.bfloat16)
        ah_ref[:, pl.ds(i * _RB, _RB)] = hi
        al_ref[:, pl.ds(i * _RB, _RB)] = (
            lb - hi.astype(jnp.float32)).astype(jnp.bfloat16)
        return carry

    jax.lax.fori_loop(0, _NRB, pass2, 0, unroll=False)

    def lmul(x_val, w):
        xh = x_val.astype(jnp.bfloat16)
        xl = (x_val - xh.astype(jnp.float32)).astype(jnp.bfloat16)
        def body(i, carry):
            hb = ah_ref[pl.ds(i * _RB, _RB), :]
            lbk = al_ref[pl.ds(i * _RB, _RB), :]
            acc = (_dot(hb, xh, None) + _dot(hb, xl, None)
                   + _dot(lbk, xh, None))
            t_ref[pl.ds(i * _RB, _RB), pl.ds(0, w)] = acc
            return carry
        jax.lax.fori_loop(0, _NRB, body, 0, unroll=False)
        return t_ref[:, pl.ds(0, w)]

    def sq_mm(src_ref, dst_ref, scale):
        def body_i(i, c0):
            def body_j(j, c1):
                acc = jnp.zeros((_SB, _SB), jnp.float32)
                for k in range(_NSB):
                    a = src_ref[pl.ds(i * _SB, _SB), pl.ds(k * _SB, _SB)]
                    b = src_ref[pl.ds(k * _SB, _SB), pl.ds(j * _SB, _SB)]
                    acc = acc + jax.lax.dot_general(
                        a.astype(jnp.bfloat16), b.astype(jnp.bfloat16),
                        (((1,), (0,)), ((), ())),
                        preferred_element_type=jnp.float32)
                dst_ref[pl.ds(i * _SB, _SB), pl.ds(j * _SB, _SB)] = (
                    scale * acc).astype(dst_ref.dtype)
                return c1
            jax.lax.fori_loop(0, _NSB, body_j, 0, unroll=False)
            return c0
        jax.lax.fori_loop(0, _NSB, body_i, 0, unroll=False)

    def mv(src_ref, x_val):
        xb16 = x_val.astype(jnp.bfloat16)
        def body(i, carry):
            ab = src_ref[pl.ds(i * _RB, _RB), :]
            t_ref[pl.ds(i * _RB, _RB), pl.ds(0, 1)] = jax.lax.dot_general(
                ab.astype(jnp.bfloat16), xb16, (((1,), (0,)), ((), ())),
                preferred_element_type=jnp.float32)
            return carry
        jax.lax.fori_loop(0, _NRB, body, 0, unroll=False)
        return t_ref[:, pl.ds(0, 1)]

    def normed(x_val):
        return x_val / jnp.sqrt(jnp.sum(x_val * x_val))

    u = jnp.full((_V, 1), 1.0 / math.sqrt(float(_V)), dtype=jnp.float32)
    sq_mm(ah_ref, m_ref, 0.25)
    sq_mm(m_ref, s_ref, 1.0)
    u = normed(mv(s_ref, u))
    sq_mm(s_ref, m_ref, 1.0)
    sq_mm(m_ref, s_ref, 1.0)
    sq_mm(s_ref, m_ref, 1.0)
    u = normed(mv(m_ref, u))
    u = normed(mv(m_ref, u))
    u = normed(mv(m_ref, u))
    v = u
    lmax = jnp.sum(v * lmul(v, 1))
    alpha = 2.0 / lmax

    def lr_mul(x_val, w):
        return alpha * lmul(x_val, w) - x_val

    def cheby(x0, ws_ref, bias, k_order, w):
        y = bias + _dot(x0, ws_ref[0])
        xkm2, xkm1 = x0, None
        for k in range(1, k_order):
            if k == 1:
                xk = lr_mul(x0, w)
            else:
                xk = 2.0 * lr_mul(xkm1, w) - xkm2
                xkm2 = xkm1
            xkm1 = xk
            y = y + _dot(xk, ws_ref[k])
        return y

    x0 = jnp.transpose(xb)
    y1 = jax.nn.relu(cheby(x0, w1_ref, b1_ref[...], _K1, _DP))
    y2 = jax.nn.relu(cheby(y1, w2_ref, b2_ref[...], _K2, _F1))

    for i in range(_NRB):
        y2b = y2[i * _RB:(i + 1) * _RB, :]
        e1 = jax.nn.relu(_dot(y2b, wf1_ref[...]) + bf1_ref[...])
        e2 = jax.nn.relu(_dot(e1, wf2_ref[...]) + bf2_ref[...])
        o_ref[0, pl.ds(i * _RB, _RB), :] = e2


@jax.jit
def kernel(x, W1, b1, W2, b2, Wf1, bf1, Wf2, bf2):
    xp = jnp.concatenate(
        [x, jnp.zeros((_B, _DP - _D, _V), jnp.float32)], axis=1)
    w1s = jnp.transpose(W1.reshape(_F1, _D, _K1), (2, 1, 0))
    w1s = jnp.concatenate(
        [w1s, jnp.zeros((_K1, _DP - _D, _F1), jnp.float32)], axis=1)
    w2s = jnp.transpose(W2.reshape(_F2, _F1, _K2), (2, 1, 0))
    wf1t = jnp.transpose(Wf1)
    wf2t = jnp.transpose(Wf2)
    b1r = b1.reshape(1, _F1)
    b2r = b2.reshape(1, _F2)
    bf1r = bf1.reshape(1, _FEAT1)
    bf2r = bf2.reshape(1, _FEAT2)

    full = lambda shp: pl.BlockSpec(shp, lambda b: (0,) * len(shp))
    out = pl.pallas_call(
        _gcn_kernel,
        grid=(_B,),
        in_specs=[
            pl.BlockSpec((1, _DP, _V), lambda b: (b, 0, 0)),
            full(w1s.shape), full(b1r.shape),
            full(w2s.shape), full(b2r.shape),
            full(wf1t.shape), full(bf1r.shape),
            full(wf2t.shape), full(bf2r.shape),
        ],
        out_specs=pl.BlockSpec((1, _V, _FEAT2), lambda b: (b, 0, 0)),
        out_shape=jax.ShapeDtypeStruct((_B, _V, _FEAT2), jnp.float32),
        scratch_shapes=[
            pltpu.VMEM((_V, _V), jnp.float32),
            pltpu.VMEM((_V, _V), jnp.bfloat16),
            pltpu.VMEM((_V, _V), jnp.bfloat16),
            pltpu.VMEM((_V, _V), jnp.bfloat16),
            pltpu.VMEM((_V, _F1), jnp.float32),
        ],
    )(xp, w1s, b1r, w2s, b2r, wf1t, bf1r, wf2t, bf2r)
    return out
